# Initial kernel scaffold; baseline (speedup 1.0000x reference)
import functools

import jax
import jax.numpy as jnp
from jax import lax
from jax.experimental import pallas as pl
from jax.experimental.pallas import tpu as pltpu
from jax.experimental.pallas import tpu_sc as plsc

N = 10000
D = 128
DH = 64
K = 10
A_P = 0.5
B_P = 0.5

NSC = 2
NT = 16
CH = 128
NCH = 160
EP = NT * NCH * CH
NCH_DEG = 80
SLAB = 632
NPAD = NT * SLAB
TRASH = N

_mesh = plsc.VectorSubcoreMesh(core_axis_name="c", subcore_axis_name="s")


def _deg_body(sidx_hbm, zeros_hbm, out_hbm, idx_v, ones_v, acc_sh, dsem):
    c = lax.axis_index("c")
    s = lax.axis_index("s")
    w = c * NT + s
    pltpu.sync_copy(sidx_hbm.at[w], idx_v)
    for i in range(CH // 16):
        ones_v[pl.ds(i * 16, 16)] = jnp.ones((16,), jnp.float32)
    pltpu.sync_copy(zeros_hbm.at[pl.ds(s * SLAB, SLAB)],
                    acc_sh.at[pl.ds(s * SLAB, SLAB)])
    plsc.subcore_barrier()

    grp = 16
    def fire_drain(g, _):
        for i in range(grp):
            pltpu.async_copy(ones_v, acc_sh.at[idx_v.at[g * grp + i]], dsem,
                             add=True)
        for i in range(grp):
            pltpu.make_async_copy(ones_v, acc_sh.at[idx_v.at[g * grp]],
                                  dsem).wait()
        return 0
    lax.fori_loop(0, NCH_DEG // grp, fire_drain, 0)

    plsc.subcore_barrier()
    pltpu.sync_copy(acc_sh.at[pl.ds(s * SLAB, SLAB)],
                    out_hbm.at[c, pl.ds(s * SLAB, SLAB)])


_deg_kernel = pl.kernel(
    _deg_body,
    out_type=jax.ShapeDtypeStruct((NSC, NPAD), jnp.float32),
    mesh=_mesh,
    scratch_types=[
        pltpu.VMEM((NCH_DEG, CH), jnp.int32),
        pltpu.VMEM((CH,), jnp.float32),
        pltpu.VMEM_SHARED((NPAD,), jnp.float32),
        pltpu.SemaphoreType.DMA,
    ],
)


def _spmm_body(g_hbm, gidx_hbm, sidx_hbm, zeros_hbm, out_hbm,
               idxg, idxs, ebufA, ebufB, acc_sh, gsA, gsB, ssA, ssB):
    c = lax.axis_index("c")
    s = lax.axis_index("s")
    pltpu.sync_copy(gidx_hbm.at[c, s], idxg)
    pltpu.sync_copy(sidx_hbm.at[s], idxs)
    pltpu.sync_copy(zeros_hbm.at[pl.ds(s * SLAB, SLAB)],
                    acc_sh.at[pl.ds(s * SLAB, SLAB)])
    plsc.subcore_barrier()

    pltpu.async_copy(g_hbm.at[idxg.at[0]], ebufA, gsA)

    def dbl(jj, _):
        j = 2 * jj
        pltpu.make_async_copy(g_hbm.at[idxg.at[j]], ebufA, gsA).wait()
        @pl.when(jj > 0)
        def _():
            pltpu.make_async_copy(ebufB, acc_sh.at[idxs.at[j]], ssB).wait()
        pltpu.async_copy(g_hbm.at[idxg.at[j + 1]], ebufB, gsB)
        pltpu.async_copy(ebufA, acc_sh.at[idxs.at[j]], ssA, add=True)
        pltpu.make_async_copy(g_hbm.at[idxg.at[j + 1]], ebufB, gsB).wait()
        pltpu.make_async_copy(ebufA, acc_sh.at[idxs.at[j]], ssA).wait()
        @pl.when(jj < NCH // 2 - 1)
        def _():
            pltpu.async_copy(g_hbm.at[idxg.at[j + 2]], ebufA, gsA)
        pltpu.async_copy(ebufB, acc_sh.at[idxs.at[j + 1]], ssB, add=True)
        return 0
    lax.fori_loop(0, NCH // 2, dbl, 0)
    pltpu.make_async_copy(ebufB, acc_sh.at[idxs.at[0]], ssB).wait()

    plsc.subcore_barrier()
    pltpu.sync_copy(acc_sh.at[pl.ds(s * SLAB, SLAB)],
                    out_hbm.at[c, pl.ds(s * SLAB, SLAB)])


_spmm_kernel = pl.kernel(
    _spmm_body,
    out_type=jax.ShapeDtypeStruct((NSC, NPAD, DH), jnp.float32),
    mesh=_mesh,
    scratch_types=[
        pltpu.VMEM((NCH, CH), jnp.int32),
        pltpu.VMEM((NCH, CH), jnp.int32),
        pltpu.VMEM((CH, DH), jnp.float32),
        pltpu.VMEM((CH, DH), jnp.float32),
        pltpu.VMEM_SHARED((NPAD, DH), jnp.float32),
        pltpu.SemaphoreType.DMA,
        pltpu.SemaphoreType.DMA,
        pltpu.SemaphoreType.DMA,
        pltpu.SemaphoreType.DMA,
    ],
)


_BN = 632


def _pre_body(deg_ref, xt_ref, alpha_ref, dis_ref, g0_ref, out0_ref):
    deg = deg_ref[0, :] + deg_ref[1, :]
    dis = jnp.where(deg > 0, lax.rsqrt(deg), jnp.float32(0.0))
    xt = xt_ref[...]
    dis_ref[...] = dis[:, None]
    g0_ref[...] = xt * dis[None, :, None]
    out0_ref[...] = alpha_ref[0] * xt


_pre_kernel = pl.pallas_call(
    _pre_body,
    grid=(NPAD // _BN,),
    in_specs=[
        pl.BlockSpec((NSC, _BN), lambda i: (0, i)),
        pl.BlockSpec((NSC, _BN, DH), lambda i: (0, i, 0)),
        pl.BlockSpec(memory_space=pltpu.SMEM),
    ],
    out_specs=[
        pl.BlockSpec((_BN, 1), lambda i: (i, 0)),
        pl.BlockSpec((NSC, _BN, DH), lambda i: (0, i, 0)),
        pl.BlockSpec((NSC, _BN, DH), lambda i: (0, i, 0)),
    ],
    out_shape=[
        jax.ShapeDtypeStruct((NPAD, 1), jnp.float32),
        jax.ShapeDtypeStruct((NSC, NPAD, DH), jnp.float32),
        jax.ShapeDtypeStruct((NSC, NPAD, DH), jnp.float32),
    ],
)


def _thetas(k):
    ab = A_P + B_P
    if k == 1:
        return -(ab + 2.0) / 2.0, (A_P - B_P) / 2.0, 0.0
    th_k = (2 * k + ab) * (2 * k + ab - 1) / (2 * k * (k + ab))
    th_kp = (2 * k + ab - 1) * (A_P ** 2 - B_P ** 2) / (
        2 * k * (k + ab) * (2 * k + ab - 2))
    th_kpp = (k + A_P - 1) * (k + B_P - 1) * (2 * k + ab) / (
        k * (k + ab) * (2 * k + ab - 2))
    return -th_k, th_kp, -th_kpp


def _dense_body(k, s_ref, pm1_ref, pm2_ref, oin_ref, dis_ref,
                beta_ref, gamma_ref, pk_ref, g_ref, oout_ref):
    t_lap, t_m1, t_m2 = _thetas(k)
    dis = dis_ref[...][None, :, :]
    pk = t_lap * (dis * s_ref[...])
    if t_m1 != 0.0:
        pk = pk + t_m1 * pm1_ref[...]
    if t_m2 != 0.0:
        pk = pk + t_m2 * pm2_ref[...]
    gp = beta_ref[k]
    for i in range(k):
        gp = gp * gamma_ref[i]
    pk_ref[...] = pk
    g_ref[...] = dis * pk
    oout_ref[...] = oin_ref[...] + gp * pk


_big_spec = pl.BlockSpec((NSC, _BN, DH), lambda i: (0, i, 0))


def _make_dense(k):
    return pl.pallas_call(
        functools.partial(_dense_body, k),
        grid=(NPAD // _BN,),
        in_specs=[
            _big_spec, _big_spec, _big_spec, _big_spec,
            pl.BlockSpec((_BN, 1), lambda i: (i, 0)),
            pl.BlockSpec(memory_space=pltpu.SMEM),
            pl.BlockSpec(memory_space=pltpu.SMEM),
        ],
        out_specs=[_big_spec, _big_spec, _big_spec],
        out_shape=[
            jax.ShapeDtypeStruct((NSC, NPAD, DH), jnp.float32),
            jax.ShapeDtypeStruct((NSC, NPAD, DH), jnp.float32),
            jax.ShapeDtypeStruct((NSC, NPAD, DH), jnp.float32),
        ],
        input_output_aliases={2: 0, 0: 1, 3: 2},
    )


_dense_kernels = [None] + [_make_dense(k) for k in range(1, K + 1)]


def kernel(x, edge_index, alpha, gamma, beta):
    E = edge_index.shape[1]
    row = edge_index[0].astype(jnp.int32)
    col = edge_index[1].astype(jnp.int32)
    pad = EP - E
    rows_p = jnp.concatenate([row, jnp.full((pad,), TRASH, jnp.int32)])
    cols_p = jnp.concatenate([col, jnp.zeros((pad,), jnp.int32)])

    gidx = jnp.stack([cols_p, cols_p + NPAD]).reshape(NSC, NT, NCH, CH)
    sidx = rows_p.reshape(NT, NCH, CH)
    sidx_deg = rows_p.reshape(NSC * NT, NCH_DEG, CH)

    z1 = jnp.zeros((NPAD,), jnp.float32)
    z2 = jnp.zeros((NPAD, DH), jnp.float32)

    xt = jnp.stack([x[:, :DH], x[:, DH:]])
    xt = jnp.concatenate(
        [xt, jnp.zeros((NSC, NPAD - N, DH), jnp.float32)], axis=1)

    deg2 = _deg_kernel(sidx_deg, z1)
    dis, g, out = _pre_kernel(deg2, xt, alpha)

    pm1, pm2 = xt, xt
    for k in range(1, K + 1):
        s_k = _spmm_kernel(g.reshape(NSC * NPAD, DH), gidx, sidx, z2)
        pk, g, out = _dense_kernels[k](s_k, pm1, pm2, out, dis, beta, gamma)
        pm2, pm1 = pm1, pk

    return jnp.concatenate([out[0, :N], out[1, :N]], axis=1)


# speedup vs baseline: 2.5050x; 2.5050x over previous
import functools

import jax
import jax.numpy as jnp
from jax import lax
from jax.experimental import pallas as pl
from jax.experimental.pallas import tpu as pltpu
from jax.experimental.pallas import tpu_sc as plsc

N = 10000
D = 128
K = 10
A_P = 0.5
B_P = 0.5

NSC = 2
NT = 16
CH = 128
NCH = 80
EP = NSC * NT * NCH * CH
SLAB = 640
NPAD = NT * SLAB
TRASH = N
ZCH = 128
RING = 4

_mesh = plsc.VectorSubcoreMesh(core_axis_name="c", subcore_axis_name="s")


def _deg_body(sidx_hbm, zeros_hbm, out_hbm, idx_v, ones_v, zbuf, acc_sh, dsem):
    c = lax.axis_index("c")
    s = lax.axis_index("s")
    pltpu.sync_copy(sidx_hbm.at[c, s], idx_v)
    for i in range(CH // 16):
        ones_v[pl.ds(i * 16, 16)] = jnp.ones((16,), jnp.float32)
    pltpu.sync_copy(zeros_hbm.at[pl.ds(s * SLAB, SLAB)], zbuf)
    pltpu.sync_copy(zbuf, acc_sh.at[pl.ds(s * SLAB, SLAB)])
    plsc.subcore_barrier()

    grp = 16
    def fire_drain(g, _):
        for i in range(grp):
            pltpu.async_copy(ones_v, acc_sh.at[idx_v.at[g * grp + i]], dsem,
                             add=True)
        for i in range(grp):
            pltpu.make_async_copy(ones_v, acc_sh.at[idx_v.at[g * grp]],
                                  dsem).wait()
        return 0
    lax.fori_loop(0, NCH // grp, fire_drain, 0)

    plsc.subcore_barrier()
    pltpu.sync_copy(acc_sh.at[pl.ds(s * SLAB, SLAB)], zbuf)
    pltpu.sync_copy(zbuf, out_hbm.at[pl.ds(c * NPAD + s * SLAB, SLAB)])


_deg_kernel = pl.kernel(
    _deg_body,
    out_type=jax.ShapeDtypeStruct((NSC * NPAD,), jnp.float32),
    mesh=_mesh,
    scratch_types=[
        pltpu.VMEM((NCH, CH), jnp.int32),
        pltpu.VMEM((CH,), jnp.float32),
        pltpu.VMEM((SLAB,), jnp.float32),
        pltpu.VMEM_SHARED((NPAD,), jnp.float32),
        pltpu.SemaphoreType.DMA,
    ],
)


def _spmm_body(g_hbm, gidx_hbm, sidx_hbm, zeros_hbm, out_hbm,
               ring, idxs, ebufA, ebufB, acc_sh, gsA, gsB, ssA, ssB,
               isA, isB):
    c = lax.axis_index("c")
    s = lax.axis_index("s")
    w = c * NT + s
    gbase = w * (NCH * CH)
    pltpu.sync_copy(sidx_hbm.at[c, s], idxs)
    for r in range(RING - 1):
        pltpu.sync_copy(gidx_hbm.at[pl.ds(gbase + r * CH, CH)], ring.at[r])
    pltpu.sync_copy(zeros_hbm, ebufA)
    for i in range(SLAB // ZCH):
        pltpu.async_copy(ebufA, acc_sh.at[pl.ds(s * SLAB + i * ZCH, ZCH)], gsA)
    for i in range(SLAB // ZCH):
        pltpu.make_async_copy(ebufA, acc_sh.at[pl.ds(s * SLAB, ZCH)],
                              gsA).wait()
    plsc.subcore_barrier()

    pltpu.async_copy(g_hbm.at[ring.at[0]], ebufA, gsA)

    def halfstep(j, ebuf_cur, ebuf_nxt, gs_cur, gs_nxt, ss_cur, ss_nxt,
                 isem, skip_sw, skip_iw, last):
        pltpu.make_async_copy(g_hbm.at[ring.at[0]], ebuf_cur, gs_cur).wait()
        @pl.when(jnp.logical_not(skip_sw))
        def _():
            pltpu.make_async_copy(ebuf_nxt, acc_sh.at[idxs.at[0]],
                                  ss_nxt).wait()
        @pl.when(jnp.logical_and(jnp.logical_not(skip_iw), j + 1 < NCH))
        def _():
            pltpu.make_async_copy(gidx_hbm.at[pl.ds(gbase, CH)], ring.at[0],
                                  isem).wait()
        @pl.when(jnp.logical_not(last))
        def _():
            pltpu.async_copy(g_hbm.at[ring.at[(j + 1) % RING]], ebuf_nxt,
                             gs_nxt)
        pltpu.async_copy(ebuf_cur, acc_sh.at[idxs.at[j]], ss_cur, add=True)
        @pl.when(j + RING - 1 < NCH)
        def _():
            pltpu.async_copy(
                gidx_hbm.at[pl.ds(gbase + (j + RING - 1) * CH, CH)],
                ring.at[(j + RING - 1) % RING], isem)

    def dbl(jj, _):
        j = 2 * jj
        halfstep(j, ebufA, ebufB, gsA, gsB, ssA, ssB, isB,
                 jj == 0, jj == 0, jnp.bool_(False))
        halfstep(j + 1, ebufB, ebufA, gsB, gsA, ssB, ssA, isA,
                 jnp.bool_(False), jj == 0, jj == NCH // 2 - 1)
        return 0
    lax.fori_loop(0, NCH // 2, dbl, 0)
    pltpu.make_async_copy(ebufB, acc_sh.at[idxs.at[0]], ssB).wait()

    plsc.subcore_barrier()
    nwb = SLAB // ZCH
    for i in range(nwb):
        buf = ebufA if i % 2 == 0 else ebufB
        sem = gsA if i % 2 == 0 else gsB
        if i >= 2:
            pltpu.make_async_copy(buf, out_hbm.at[c, pl.ds(s * SLAB, ZCH)],
                                  sem).wait()
        pltpu.sync_copy(acc_sh.at[pl.ds(s * SLAB + i * ZCH, ZCH)], buf)
        pltpu.async_copy(buf, out_hbm.at[c, pl.ds(s * SLAB + i * ZCH, ZCH)],
                         sem)
    pltpu.make_async_copy(ebufA, out_hbm.at[c, pl.ds(s * SLAB, ZCH)],
                          gsA).wait()
    pltpu.make_async_copy(ebufB, out_hbm.at[c, pl.ds(s * SLAB, ZCH)],
                          gsB).wait()


_spmm_kernel = pl.kernel(
    _spmm_body,
    out_type=jax.ShapeDtypeStruct((NSC, NPAD, D), jnp.float32),
    mesh=_mesh,
    scratch_types=[
        pltpu.VMEM((RING, CH), jnp.int32),
        pltpu.VMEM((NCH, CH), jnp.int32),
        pltpu.VMEM((CH, D), jnp.float32),
        pltpu.VMEM((CH, D), jnp.float32),
        pltpu.VMEM_SHARED((NPAD, D), jnp.float32),
        pltpu.SemaphoreType.DMA,
        pltpu.SemaphoreType.DMA,
        pltpu.SemaphoreType.DMA,
        pltpu.SemaphoreType.DMA,
        pltpu.SemaphoreType.DMA,
        pltpu.SemaphoreType.DMA,
    ],
)


_BN = 640


def _pre_body(degT_ref, xt_ref, alpha_ref, dis_ref, g0_ref, out0_ref):
    deg = degT_ref[:, 0:1] + degT_ref[:, 1:2]
    dis = jnp.where(deg > 0, lax.rsqrt(deg), jnp.float32(0.0))
    xt = xt_ref[...]
    dis_ref[...] = dis
    g0_ref[...] = xt * dis
    out0_ref[...] = alpha_ref[0] * xt


_row_spec = pl.BlockSpec((_BN, D), lambda i: (i, 0))
_dis_spec = pl.BlockSpec((_BN, 1), lambda i: (i, 0))

_pre_kernel = pl.pallas_call(
    _pre_body,
    grid=(NPAD // _BN,),
    in_specs=[
        pl.BlockSpec((_BN, NSC), lambda i: (i, 0)),
        _row_spec,
        pl.BlockSpec(memory_space=pltpu.SMEM),
    ],
    out_specs=[_dis_spec, _row_spec, _row_spec],
    out_shape=[
        jax.ShapeDtypeStruct((NPAD, 1), jnp.float32),
        jax.ShapeDtypeStruct((NPAD, D), jnp.float32),
        jax.ShapeDtypeStruct((NPAD, D), jnp.float32),
    ],
)


def _thetas(k):
    ab = A_P + B_P
    if k == 1:
        return -(ab + 2.0) / 2.0, (A_P - B_P) / 2.0, 0.0
    th_k = (2 * k + ab) * (2 * k + ab - 1) / (2 * k * (k + ab))
    th_kp = (2 * k + ab - 1) * (A_P ** 2 - B_P ** 2) / (
        2 * k * (k + ab) * (2 * k + ab - 2))
    th_kpp = (k + A_P - 1) * (k + B_P - 1) * (2 * k + ab) / (
        k * (k + ab) * (2 * k + ab - 2))
    return -th_k, th_kp, -th_kpp


def _dense_body(k, s_ref, pm1_ref, pm2_ref, oin_ref, dis_ref,
                beta_ref, gamma_ref, pk_ref, g_ref, oout_ref):
    t_lap, t_m1, t_m2 = _thetas(k)
    dis = dis_ref[...]
    s = s_ref[0] + s_ref[1]
    pk = t_lap * (dis * s)
    if t_m1 != 0.0:
        pk = pk + t_m1 * pm1_ref[...]
    if t_m2 != 0.0:
        pk = pk + t_m2 * pm2_ref[...]
    gp = beta_ref[k]
    for i in range(k):
        gp = gp * gamma_ref[i]
    pk_ref[...] = pk
    g_ref[...] = dis * pk
    oout_ref[...] = oin_ref[...] + gp * pk


def _make_dense(k):
    return pl.pallas_call(
        functools.partial(_dense_body, k),
        grid=(NPAD // _BN,),
        in_specs=[
            pl.BlockSpec((NSC, _BN, D), lambda i: (0, i, 0)),
            _row_spec, _row_spec, _row_spec, _dis_spec,
            pl.BlockSpec(memory_space=pltpu.SMEM),
            pl.BlockSpec(memory_space=pltpu.SMEM),
        ],
        out_specs=[_row_spec, _row_spec, _row_spec],
        out_shape=[
            jax.ShapeDtypeStruct((NPAD, D), jnp.float32),
            jax.ShapeDtypeStruct((NPAD, D), jnp.float32),
            jax.ShapeDtypeStruct((NPAD, D), jnp.float32),
        ],
        input_output_aliases={2: 0, 3: 2},
    )


_dense_kernels = [None] + [_make_dense(k) for k in range(1, K + 1)]


def kernel(x, edge_index, alpha, gamma, beta):
    E = edge_index.shape[1]
    row = edge_index[0].astype(jnp.int32)
    col = edge_index[1].astype(jnp.int32)
    pad = EP - E
    rows_p = jnp.concatenate([row, jnp.full((pad,), TRASH, jnp.int32)])
    cols_p = jnp.concatenate([col, jnp.zeros((pad,), jnp.int32)])
    cols_p, rows_p = lax.sort_key_val(cols_p, rows_p)

    gidx = cols_p
    sidx = rows_p.reshape(NSC, NT, NCH, CH)

    z1 = jnp.zeros((NPAD,), jnp.float32)
    z2 = jnp.zeros((ZCH, D), jnp.float32)

    xt = jnp.concatenate([x, jnp.zeros((NPAD - N, D), jnp.float32)])

    deg2 = _deg_kernel(sidx, z1).reshape(NSC, NPAD)
    dis, g, out = _pre_kernel(deg2.T, xt, alpha)

    pm1, pm2 = xt, xt
    for k in range(1, K + 1):
        s_k = _spmm_kernel(g, gidx, sidx, z2)
        pk, g, out = _dense_kernels[k](s_k, pm1, pm2, out, dis, beta, gamma)
        pm2, pm1 = pm1, pk

    return out[:N]


# speedup vs baseline: 3.4961x; 1.3956x over previous
import functools

import jax
import jax.numpy as jnp
from jax import lax
from jax.experimental import pallas as pl
from jax.experimental.pallas import tpu as pltpu
from jax.experimental.pallas import tpu_sc as plsc

N = 10000
D = 128
K = 10
A_P = 0.5
B_P = 0.5

NSC = 2
NT = 16
CH = 128
NCH = 80
EP = NSC * NT * NCH * CH
SLAB = 640
NPAD = NT * SLAB
TRASH = N
ZCH = 128
RING = 4

_mesh = plsc.VectorSubcoreMesh(core_axis_name="c", subcore_axis_name="s")


def _deg_body(sidx_hbm, zeros_hbm, out_hbm, idx_v, ones_v, zbuf, acc_sh, dsem):
    c = lax.axis_index("c")
    s = lax.axis_index("s")
    pltpu.sync_copy(sidx_hbm.at[c, s], idx_v)
    for i in range(CH // 16):
        ones_v[pl.ds(i * 16, 16)] = jnp.ones((16,), jnp.float32)
    pltpu.sync_copy(zeros_hbm.at[pl.ds(s * SLAB, SLAB)], zbuf)
    pltpu.sync_copy(zbuf, acc_sh.at[pl.ds(s * SLAB, SLAB)])
    plsc.subcore_barrier()

    grp = 16
    def fire_drain(g, _):
        for i in range(grp):
            pltpu.async_copy(ones_v, acc_sh.at[idx_v.at[g * grp + i]], dsem,
                             add=True)
        for i in range(grp):
            pltpu.make_async_copy(ones_v, acc_sh.at[idx_v.at[g * grp]],
                                  dsem).wait()
        return 0
    lax.fori_loop(0, NCH // grp, fire_drain, 0)

    plsc.subcore_barrier()
    pltpu.sync_copy(acc_sh.at[pl.ds(s * SLAB, SLAB)], zbuf)
    pltpu.sync_copy(zbuf, out_hbm.at[pl.ds(c * NPAD + s * SLAB, SLAB)])


_deg_kernel = pl.kernel(
    _deg_body,
    out_type=jax.ShapeDtypeStruct((NSC * NPAD,), jnp.float32),
    mesh=_mesh,
    scratch_types=[
        pltpu.VMEM((NCH, CH), jnp.int32),
        pltpu.VMEM((CH,), jnp.float32),
        pltpu.VMEM((SLAB,), jnp.float32),
        pltpu.VMEM_SHARED((NPAD,), jnp.float32),
        pltpu.SemaphoreType.DMA,
    ],
)


def _spmm_body(g_hbm, gidx_hbm, sidx_hbm, zeros_hbm, out_hbm,
               ring, idxs, ebufA, ebufB, acc_sh, gsA, gsB, ssA, ssB,
               isA, isB):
    c = lax.axis_index("c")
    s = lax.axis_index("s")
    w = c * NT + s
    gbase = w * (NCH * CH)
    pltpu.sync_copy(sidx_hbm.at[c, s], idxs)
    for r in range(RING - 1):
        pltpu.sync_copy(gidx_hbm.at[pl.ds(gbase + r * CH, CH)], ring.at[r])
    pltpu.sync_copy(zeros_hbm, ebufA)
    for i in range(SLAB // ZCH):
        pltpu.async_copy(ebufA, acc_sh.at[pl.ds(s * SLAB + i * ZCH, ZCH)], gsA)
    for i in range(SLAB // ZCH):
        pltpu.make_async_copy(ebufA, acc_sh.at[pl.ds(s * SLAB, ZCH)],
                              gsA).wait()
    plsc.subcore_barrier()

    pltpu.async_copy(g_hbm.at[ring.at[0]], ebufA, gsA)

    def halfstep(j, ebuf_cur, ebuf_nxt, gs_cur, gs_nxt, ss_cur, ss_nxt,
                 isem, skip_sw, skip_iw, last):
        pltpu.make_async_copy(g_hbm.at[ring.at[0]], ebuf_cur, gs_cur).wait()
        @pl.when(jnp.logical_not(skip_sw))
        def _():
            pltpu.make_async_copy(ebuf_nxt, acc_sh.at[idxs.at[0]],
                                  ss_nxt).wait()
        @pl.when(jnp.logical_and(jnp.logical_not(skip_iw), j + 1 < NCH))
        def _():
            pltpu.make_async_copy(gidx_hbm.at[pl.ds(gbase, CH)], ring.at[0],
                                  isem).wait()
        @pl.when(jnp.logical_not(last))
        def _():
            pltpu.async_copy(g_hbm.at[ring.at[(j + 1) % RING]], ebuf_nxt,
                             gs_nxt)
        pltpu.async_copy(ebuf_cur, acc_sh.at[idxs.at[j]], ss_cur, add=True)
        @pl.when(j + RING - 1 < NCH)
        def _():
            pltpu.async_copy(
                gidx_hbm.at[pl.ds(gbase + (j + RING - 1) * CH, CH)],
                ring.at[(j + RING - 1) % RING], isem)

    def dbl(jj, _):
        j = 2 * jj
        halfstep(j, ebufA, ebufB, gsA, gsB, ssA, ssB, isB,
                 jj == 0, jj == 0, jnp.bool_(False))
        halfstep(j + 1, ebufB, ebufA, gsB, gsA, ssB, ssA, isA,
                 jnp.bool_(False), jj == 0, jj == NCH // 2 - 1)
        return 0
    lax.fori_loop(0, NCH // 2, dbl, 0)
    pltpu.make_async_copy(ebufB, acc_sh.at[idxs.at[0]], ssB).wait()

    plsc.subcore_barrier()
    nwb = SLAB // ZCH
    for i in range(nwb):
        buf = ebufA if i % 2 == 0 else ebufB
        sem = gsA if i % 2 == 0 else gsB
        if i >= 2:
            pltpu.make_async_copy(buf, out_hbm.at[c, pl.ds(s * SLAB, ZCH)],
                                  sem).wait()
        pltpu.sync_copy(acc_sh.at[pl.ds(s * SLAB + i * ZCH, ZCH)], buf)
        pltpu.async_copy(buf, out_hbm.at[c, pl.ds(s * SLAB + i * ZCH, ZCH)],
                         sem)
    pltpu.make_async_copy(ebufA, out_hbm.at[c, pl.ds(s * SLAB, ZCH)],
                          gsA).wait()
    pltpu.make_async_copy(ebufB, out_hbm.at[c, pl.ds(s * SLAB, ZCH)],
                          gsB).wait()


_spmm_kernel = pl.kernel(
    _spmm_body,
    out_type=jax.ShapeDtypeStruct((NSC, NPAD, D), jnp.float32),
    mesh=_mesh,
    scratch_types=[
        pltpu.VMEM((RING, CH), jnp.int32),
        pltpu.VMEM((NCH, CH), jnp.int32),
        pltpu.VMEM((CH, D), jnp.float32),
        pltpu.VMEM((CH, D), jnp.float32),
        pltpu.VMEM_SHARED((NPAD, D), jnp.float32),
        pltpu.SemaphoreType.DMA,
        pltpu.SemaphoreType.DMA,
        pltpu.SemaphoreType.DMA,
        pltpu.SemaphoreType.DMA,
        pltpu.SemaphoreType.DMA,
        pltpu.SemaphoreType.DMA,
    ],
)


_BN = 640


def _pre_body(degT_ref, xt_ref, alpha_ref, dis_ref, g0_ref, out0_ref):
    deg = degT_ref[:, 0:1] + degT_ref[:, 1:2]
    dis = jnp.where(deg > 0, lax.rsqrt(deg), jnp.float32(0.0))
    xt = xt_ref[...]
    dis_ref[...] = dis
    g0_ref[...] = xt * dis
    out0_ref[...] = alpha_ref[0] * xt


_row_spec = pl.BlockSpec((_BN, D), lambda i: (i, 0))
_dis_spec = pl.BlockSpec((_BN, 1), lambda i: (i, 0))

_pre_kernel = pl.pallas_call(
    _pre_body,
    grid=(NPAD // _BN,),
    in_specs=[
        pl.BlockSpec((_BN, NSC), lambda i: (i, 0)),
        _row_spec,
        pl.BlockSpec(memory_space=pltpu.SMEM),
    ],
    out_specs=[_dis_spec, _row_spec, _row_spec],
    out_shape=[
        jax.ShapeDtypeStruct((NPAD, 1), jnp.float32),
        jax.ShapeDtypeStruct((NPAD, D), jnp.float32),
        jax.ShapeDtypeStruct((NPAD, D), jnp.float32),
    ],
)


def _thetas(k):
    ab = A_P + B_P
    if k == 1:
        return -(ab + 2.0) / 2.0, (A_P - B_P) / 2.0, 0.0
    th_k = (2 * k + ab) * (2 * k + ab - 1) / (2 * k * (k + ab))
    th_kp = (2 * k + ab - 1) * (A_P ** 2 - B_P ** 2) / (
        2 * k * (k + ab) * (2 * k + ab - 2))
    th_kpp = (k + A_P - 1) * (k + B_P - 1) * (2 * k + ab) / (
        k * (k + ab) * (2 * k + ab - 2))
    return -th_k, th_kp, -th_kpp


def _dense_body(k, s_ref, pm1_ref, pm2_ref, oin_ref, dis_ref,
                beta_ref, gamma_ref, pk_ref, g_ref, oout_ref):
    t_lap, t_m1, t_m2 = _thetas(k)
    dis = dis_ref[...]
    s = s_ref[0] + s_ref[1]
    pk = t_lap * (dis * s)
    if t_m1 != 0.0:
        pk = pk + t_m1 * pm1_ref[...]
    if t_m2 != 0.0:
        pk = pk + t_m2 * pm2_ref[...]
    gp = beta_ref[k]
    for i in range(k):
        gp = gp * gamma_ref[i]
    pk_ref[...] = pk
    g_ref[...] = dis * pk
    oout_ref[...] = oin_ref[...] + gp * pk


def _make_dense(k):
    return pl.pallas_call(
        functools.partial(_dense_body, k),
        grid=(NPAD // _BN,),
        in_specs=[
            pl.BlockSpec((NSC, _BN, D), lambda i: (0, i, 0)),
            _row_spec, _row_spec, _row_spec, _dis_spec,
            pl.BlockSpec(memory_space=pltpu.SMEM),
            pl.BlockSpec(memory_space=pltpu.SMEM),
        ],
        out_specs=[_row_spec, _row_spec, _row_spec],
        out_shape=[
            jax.ShapeDtypeStruct((NPAD, D), jnp.float32),
            jax.ShapeDtypeStruct((NPAD, D), jnp.float32),
            jax.ShapeDtypeStruct((NPAD, D), jnp.float32),
        ],
        input_output_aliases={2: 0, 3: 2},
    )


_dense_kernels = [None] + [_make_dense(k) for k in range(1, K + 1)]


def kernel(x, edge_index, alpha, gamma, beta):
    E = edge_index.shape[1]
    row = edge_index[0].astype(jnp.int32)
    col = edge_index[1].astype(jnp.int32)
    pad = EP - E
    rows_p = jnp.concatenate([row, jnp.full((pad,), TRASH, jnp.int32)])
    cols_p = jnp.concatenate([col, jnp.zeros((pad,), jnp.int32)])

    gidx = cols_p
    sidx = rows_p.reshape(NSC, NT, NCH, CH)

    z1 = jnp.zeros((NPAD,), jnp.float32)
    z2 = jnp.zeros((ZCH, D), jnp.float32)

    xt = jnp.concatenate([x, jnp.zeros((NPAD - N, D), jnp.float32)])

    deg2 = _deg_kernel(sidx, z1).reshape(NSC, NPAD)
    dis, g, out = _pre_kernel(deg2.T, xt, alpha)

    pm1, pm2 = xt, xt
    for k in range(1, K + 1):
        s_k = _spmm_kernel(g, gidx, sidx, z2)
        pk, g, out = _dense_kernels[k](s_k, pm1, pm2, out, dis, beta, gamma)
        pm2, pm1 = pm1, pk

    return out[:N]
